# initial kernel scaffold (unmeasured)
import jax
import jax.numpy as jnp
from jax import lax
from jax.experimental import pallas as pl
from jax.experimental.pallas import tpu as pltpu


def kernel(
    t,
):
    def body(*refs):
        pass

    out_shape = jax.ShapeDtypeStruct(..., jnp.float32)
    return pl.pallas_call(body, out_shape=out_shape)(...)


# baseline (device time: 509207 ns/iter reference)
import jax
import jax.numpy as jnp
from jax import lax
from jax.experimental import pallas as pl
from jax.experimental.pallas import tpu as pltpu

N_DEV = 16
M_PER = 4096
N_COLS = 1024
CHUNK = M_PER // N_DEV


def _f(s):
    r = jnp.maximum(s, 0.0)
    return jnp.tanh(s) * s * s + r * r * r


def kernel(t):
    def body(x_ref, out_ref, send_ref, recv_ref, send_sem, recv_sem, credit_sem):
        my = lax.axis_index("i")
        left = lax.rem(my + N_DEV - 1, N_DEV)
        right = lax.rem(my + 1, N_DEV)

        barrier_sem = pltpu.get_barrier_semaphore()
        pl.semaphore_signal(
            barrier_sem, inc=1, device_id=(left,),
            device_id_type=pl.DeviceIdType.MESH,
        )
        pl.semaphore_signal(
            barrier_sem, inc=1, device_id=(right,),
            device_id_type=pl.DeviceIdType.MESH,
        )
        pl.semaphore_wait(barrier_sem, 2)

        def hop(wait_credit):
            rdma = pltpu.make_async_remote_copy(
                src_ref=send_ref,
                dst_ref=recv_ref,
                send_sem=send_sem,
                recv_sem=recv_sem,
                device_id=(right,),
                device_id_type=pl.DeviceIdType.MESH,
            )
            if wait_credit:
                pl.semaphore_wait(credit_sem, 1)
            rdma.start()
            rdma.wait()

        send_ref[...] = x_ref[pl.ds(my * CHUNK, CHUNK), :]
        for s in range(N_DEV - 1):
            hop(wait_credit=s > 0)
            c_recv = lax.rem(my + (2 * N_DEV - s - 1) % N_DEV, N_DEV)
            mine = x_ref[pl.ds(c_recv * CHUNK, CHUNK), :]
            if s < N_DEV - 2:
                send_ref[...] = recv_ref[...] + mine
            else:
                res = _f(recv_ref[...] + mine)
                out_ref[pl.ds(c_recv * CHUNK, CHUNK), :] = res
                send_ref[...] = res
            pl.semaphore_signal(
                credit_sem, inc=1, device_id=(left,),
                device_id_type=pl.DeviceIdType.MESH,
            )

        for s in range(N_DEV - 1):
            hop(wait_credit=True)
            c_recv = lax.rem(my + (N_DEV - s) % N_DEV, N_DEV)
            out_ref[pl.ds(c_recv * CHUNK, CHUNK), :] = recv_ref[...]
            if s < N_DEV - 2:
                send_ref[...] = recv_ref[...]
                pl.semaphore_signal(
                    credit_sem, inc=1, device_id=(left,),
                    device_id_type=pl.DeviceIdType.MESH,
                )

    return pl.pallas_call(
        body,
        out_shape=jax.ShapeDtypeStruct((M_PER, N_COLS), jnp.float32),
        in_specs=[pl.BlockSpec(memory_space=pltpu.VMEM)],
        out_specs=pl.BlockSpec(memory_space=pltpu.VMEM),
        scratch_shapes=[
            pltpu.VMEM((CHUNK, N_COLS), jnp.float32),
            pltpu.VMEM((CHUNK, N_COLS), jnp.float32),
            pltpu.SemaphoreType.DMA,
            pltpu.SemaphoreType.DMA,
            pltpu.SemaphoreType.REGULAR,
        ],
        compiler_params=pltpu.CompilerParams(collective_id=0),
    )(t)


# device time: 271594 ns/iter; 1.8749x vs baseline; 1.8749x over previous
import jax
import jax.numpy as jnp
from jax import lax
from jax.experimental import pallas as pl
from jax.experimental.pallas import tpu as pltpu

N_DEV = 16
M_PER = 4096
N_COLS = 1024
CHUNK = M_PER // N_DEV
HALF = CHUNK // 2
NSUB = 2
SUB = HALF // NSUB

_MESH = pl.DeviceIdType.MESH


def _f(s):
    r = jnp.maximum(s, 0.0)
    return jnp.tanh(s) * s * s + r * r * r


def kernel(t):
    def body(x_ref, out_ref, send_cw, recv_cw, send_ccw, recv_ccw,
             ssem_cw, rsem_cw, ssem_ccw, rsem_ccw, cred_cw, cred_ccw):
        my = lax.axis_index("i")
        left = lax.rem(my + N_DEV - 1, N_DEV)
        right = lax.rem(my + 1, N_DEV)

        barrier_sem = pltpu.get_barrier_semaphore()
        for nbr in (left, right):
            pl.semaphore_signal(barrier_sem, inc=1, device_id=(nbr,),
                                device_id_type=_MESH)
        pl.semaphore_wait(barrier_sem, 2)

        def bufs(d):
            if d == 0:
                return send_cw, recv_cw, cred_cw, right, left
            else:
                return send_ccw, recv_ccw, cred_ccw, left, right

        def rdma(d, j):
            sb, rb, _, to, _ = bufs(d)
            ss = (ssem_cw, ssem_ccw)[d]
            rs = (rsem_cw, rsem_ccw)[d]
            return pltpu.make_async_remote_copy(
                src_ref=sb.at[j], dst_ref=rb.at[j],
                send_sem=ss.at[j], recv_sem=rs.at[j],
                device_id=(to,), device_id_type=_MESH)

        def chunk(off):
            return lax.rem(my + (off % N_DEV), N_DEV)

        def rows(c, d, j):
            return pl.ds(c * CHUNK + d * HALF + j * SUB, SUB)

        LANES = [(0, 0), (1, 0), (0, 1), (1, 1)]

        for d, j in LANES:
            sb = (send_cw, send_ccw)[d]
            sb[j, :, :] = x_ref[rows(chunk(0), d, j), :]
            rdma(d, j).start()

        for s in range(N_DEV - 1):
            last = s == N_DEV - 2
            for d, j in LANES:
                sb, rb, cred, to, frm = bufs(d)
                desc = rdma(d, j)
                desc.wait_recv()
                c = chunk(-(s + 1) if d == 0 else (s + 1))
                mine = x_ref[rows(c, d, j), :]
                desc.wait_send()
                if not last:
                    sb[j, :, :] = rb[j, :, :] + mine
                else:
                    res = _f(rb[j, :, :] + mine)
                    out_ref[rows(c, d, j), :] = res
                    sb[j, :, :] = res
                pl.semaphore_signal(cred, inc=1, device_id=(frm,),
                                    device_id_type=_MESH)
                pl.semaphore_wait(cred, 1)
                rdma(d, j).start()

        for s in range(N_DEV - 1):
            last = s == N_DEV - 2
            for d, j in LANES:
                sb, rb, cred, to, frm = bufs(d)
                desc = rdma(d, j)
                desc.wait_recv()
                c = chunk(-s if d == 0 else s)
                data = rb[j, :, :]
                out_ref[rows(c, d, j), :] = data
                if not last:
                    desc.wait_send()
                    sb[j, :, :] = data
                    pl.semaphore_signal(cred, inc=1, device_id=(frm,),
                                        device_id_type=_MESH)
                    pl.semaphore_wait(cred, 1)
                    rdma(d, j).start()
                else:
                    desc.wait_send()

    return pl.pallas_call(
        body,
        out_shape=jax.ShapeDtypeStruct((M_PER, N_COLS), jnp.float32),
        in_specs=[pl.BlockSpec(memory_space=pltpu.VMEM)],
        out_specs=pl.BlockSpec(memory_space=pltpu.VMEM),
        scratch_shapes=[
            pltpu.VMEM((NSUB, SUB, N_COLS), jnp.float32),
            pltpu.VMEM((NSUB, SUB, N_COLS), jnp.float32),
            pltpu.VMEM((NSUB, SUB, N_COLS), jnp.float32),
            pltpu.VMEM((NSUB, SUB, N_COLS), jnp.float32),
            pltpu.SemaphoreType.DMA((NSUB,)),
            pltpu.SemaphoreType.DMA((NSUB,)),
            pltpu.SemaphoreType.DMA((NSUB,)),
            pltpu.SemaphoreType.DMA((NSUB,)),
            pltpu.SemaphoreType.REGULAR,
            pltpu.SemaphoreType.REGULAR,
        ],
        compiler_params=pltpu.CompilerParams(collective_id=0),
    )(t)


# device time: 188255 ns/iter; 2.7049x vs baseline; 1.4427x over previous
import jax
import jax.numpy as jnp
from jax import lax
from jax.experimental import pallas as pl
from jax.experimental.pallas import tpu as pltpu

N_DEV = 16
M_PER = 4096
N_COLS = 1024
CHUNK = M_PER // N_DEV
HALF = CHUNK // 2
NSUB = 2
SUB = HALF // NSUB

_MESH = pl.DeviceIdType.MESH
_NQ = 2 * N_DEV - 2


def _f(s):
    r = jnp.maximum(s, 0.0)
    return jnp.tanh(s) * s * s + r * r * r


def kernel(t):
    def body(x_ref, out_ref, send_cw, recv_cw, send_ccw, recv_ccw,
             ssem_cw, rsem_cw, ssem_ccw, rsem_ccw, cred_cw, cred_ccw):
        my = lax.axis_index("i")
        left = lax.rem(my + N_DEV - 1, N_DEV)
        right = lax.rem(my + 1, N_DEV)

        barrier_sem = pltpu.get_barrier_semaphore()
        for nbr in (left, right):
            pl.semaphore_signal(barrier_sem, inc=1, device_id=(nbr,),
                                device_id_type=_MESH)
        pl.semaphore_wait(barrier_sem, 2)

        def bufs(d):
            if d == 0:
                return send_cw, recv_cw, ssem_cw, rsem_cw, cred_cw, right, left
            return send_ccw, recv_ccw, ssem_ccw, rsem_ccw, cred_ccw, left, right

        def chunk(off):
            return lax.rem(my + (off % N_DEV), N_DEV)

        def rows(c, d, j):
            return pl.ds(c * CHUNK + d * HALF + j * SUB, SUB)

        def rs_rdma(d, j, q):
            sb, rb, ss, rs, _, to, _ = bufs(d)
            p = q % 2
            return pltpu.make_async_remote_copy(
                src_ref=sb.at[j, p], dst_ref=rb.at[j, p],
                send_sem=ss.at[j, p], recv_sem=rs.at[j, p],
                device_id=(to,), device_id_type=_MESH)

        def ag_rdma(d, j, q):
            _, _, ss, rs, _, to, _ = bufs(d)
            k = q - (N_DEV - 1)
            c = chunk((1 - k) if d == 0 else (k - 1))
            p = q % 2
            return pltpu.make_async_remote_copy(
                src_ref=out_ref.at[rows(c, d, j)],
                dst_ref=out_ref.at[rows(c, d, j)],
                send_sem=ss.at[j, p], recv_sem=rs.at[j, p],
                device_id=(to,), device_id_type=_MESH)

        LANES = [(0, 0), (1, 0), (0, 1), (1, 1)]

        for d, j in LANES:
            sb = (send_cw, send_ccw)[d]
            sb[j, 0, :, :] = x_ref[rows(chunk(0), d, j), :]
            rs_rdma(d, j, 0).start()

        for q in range(1, _NQ):
            r = q - 1
            for d, j in LANES:
                sb, rb, ss, rs, cred, to, frm = bufs(d)
                if r <= N_DEV - 2:
                    rs_rdma(d, j, r).wait_recv()
                else:
                    ag_rdma(d, j, r).wait_recv()

                if q <= N_DEV - 1:
                    c = chunk(-(r + 1) if d == 0 else (r + 1))
                    acc = rb[j, r % 2, :, :] + x_ref[rows(c, d, j), :]
                    nxt = rs_rdma(d, j, q) if q <= N_DEV - 2 else ag_rdma(d, j, q)
                    if q >= 2:
                        nxt.wait_send()
                    if q <= N_DEV - 2:
                        sb[j, q % 2, :, :] = acc
                    else:
                        out_ref[rows(c, d, j), :] = _f(acc)
                else:
                    nxt = ag_rdma(d, j, q)
                    nxt.wait_send()
                if r <= _NQ - 3:
                    pl.semaphore_signal(cred, inc=1, device_id=(frm,),
                                        device_id_type=_MESH)
                if q >= 2:
                    pl.semaphore_wait(cred, 1)
                nxt.start()

        for d, j in LANES:
            ag_rdma(d, j, _NQ - 1).wait_recv()
            ag_rdma(d, j, _NQ - 2).wait_send()
            ag_rdma(d, j, _NQ - 1).wait_send()

    return pl.pallas_call(
        body,
        out_shape=jax.ShapeDtypeStruct((M_PER, N_COLS), jnp.float32),
        in_specs=[pl.BlockSpec(memory_space=pltpu.VMEM)],
        out_specs=pl.BlockSpec(memory_space=pltpu.VMEM),
        scratch_shapes=[
            pltpu.VMEM((NSUB, 2, SUB, N_COLS), jnp.float32),
            pltpu.VMEM((NSUB, 2, SUB, N_COLS), jnp.float32),
            pltpu.VMEM((NSUB, 2, SUB, N_COLS), jnp.float32),
            pltpu.VMEM((NSUB, 2, SUB, N_COLS), jnp.float32),
            pltpu.SemaphoreType.DMA((NSUB, 2)),
            pltpu.SemaphoreType.DMA((NSUB, 2)),
            pltpu.SemaphoreType.DMA((NSUB, 2)),
            pltpu.SemaphoreType.DMA((NSUB, 2)),
            pltpu.SemaphoreType.REGULAR,
            pltpu.SemaphoreType.REGULAR,
        ],
        compiler_params=pltpu.CompilerParams(collective_id=0),
    )(t)
